# baseline (speedup 1.0000x reference)
import functools

import jax
import jax.numpy as jnp
from jax import lax
from jax.experimental import pallas as pl
from jax.experimental.pallas import tpu as pltpu
from jax.experimental.pallas import tpu_sc as plsc

_NC = 2
_NS = 16
_NW = _NC * _NS

_EMBED = 64
_CHUNK = 256
_NBUF = 2

_VOCAB = 1000000
_BLK = 128
_NFULL = _VOCAB // _BLK
_TAIL = _VOCAB - _NFULL * _BLK
_STG_W = 130


def _sc_relayout(tab_t, tail_t):
    jmax = 246
    n_pairs = jmax // 2

    mesh = plsc.VectorSubcoreMesh(
        core_axis_name="c", subcore_axis_name="s",
        num_cores=_NC, num_subcores=_NS)

    @functools.partial(
        pl.kernel,
        out_type=jax.ShapeDtypeStruct((_VOCAB // 2, 128), jnp.float32),
        mesh=mesh,
        scratch_types=[
            pltpu.VMEM((64, _BLK), jnp.float32),
            pltpu.VMEM((64, _BLK), jnp.float32),
            pltpu.VMEM((64, _STG_W), jnp.float32),
            pltpu.VMEM((64, _STG_W), jnp.float32),
        ] + [pltpu.SemaphoreType.DMA] * 4,
        compiler_params=pltpu.CompilerParams(
            use_tc_tiling_on_sc=True, needs_layout_passes=False),
    )
    def relayout_kernel(t_hbm, tail_hbm, out_hbm, in_a, in_b, st_a, st_b,
                        gsem_a, gsem_b, wsem_a, wsem_b):
        wid = lax.axis_index("s") * _NC + lax.axis_index("c")
        ins = (in_a, in_b)
        sts = (st_a, st_b)
        gsems = (gsem_a, gsem_b)
        wsems = (wsem_a, wsem_b)

        iota = lax.iota(jnp.int32, 16)
        half = lax.shift_right_logical(iota, 1)
        cbase = (iota & 1) * 64
        rbases = [8 * g + half for g in range(8)]

        def blk_of(j):
            return wid + _NW * j

        def fire_in(j, buf):
            pltpu.async_copy(
                t_hbm.at[:, pl.ds(blk_of(j) * _BLK, _BLK)],
                ins[buf], gsems[buf])

        def wait_in(buf):
            pltpu.make_async_copy(
                t_hbm.at[:, pl.ds(0, _BLK)], ins[buf], gsems[buf]).wait()

        def fire_out(j, buf):
            pltpu.async_copy(
                sts[buf].at[:, pl.ds(0, 128)],
                out_hbm.at[pl.ds(blk_of(j) * 64, 64)], wsems[buf])

        def wait_out(buf):
            pltpu.make_async_copy(
                sts[buf].at[:, pl.ds(0, 128)],
                out_hbm.at[pl.ds(0, 64)], wsems[buf]).wait()

        def compute(buf, ngroups):
            v_in, st = ins[buf], sts[buf]

            def frow(f, carry):
                c_idx = cbase + f
                for g in range(ngroups):
                    v = v_in[f, pl.ds(16 * g, 16)]
                    plsc.store_scatter(st, [rbases[g], c_idx], v)
                return carry

            lax.fori_loop(0, 64, frow, 0)

        def do_j(j, buf):
            valid = blk_of(j) < _NFULL
            valid_next = blk_of(j + 1) < _NFULL

            @pl.when(valid_next)
            def _():
                fire_in(j + 1, 1 - buf)

            @pl.when(valid)
            def _():
                wait_in(buf)

                @pl.when(j >= 2)
                def _():
                    wait_out(buf)

                compute(buf, 8)
                fire_out(j, buf)

        fire_in(0, 0)

        def pair(s, carry):
            do_j(2 * s, 0)
            do_j(2 * s + 1, 1)
            return carry

        lax.fori_loop(0, n_pairs, pair, 0)

        wait_out(0)
        wait_out(1)

        @pl.when(wid == _NW - 1)
        def _():
            pltpu.sync_copy(tail_hbm, in_a)

            def frow_t(f, carry):
                c_idx = cbase + f
                for g in range(_TAIL // 16):
                    v = in_a[f, pl.ds(16 * g, 16)]
                    plsc.store_scatter(st_a, [rbases[g], c_idx], v)
                return carry

            lax.fori_loop(0, 64, frow_t, 0)
            pltpu.sync_copy(
                st_a.at[pl.ds(0, _TAIL // 2), pl.ds(0, 128)],
                out_hbm.at[pl.ds(_NFULL * 64, _TAIL // 2)])

    return relayout_kernel(tab_t, tail_t)


def _sc_gather(idx_flat, table, *, n_rows):
    rows_per_w = n_rows // _NW
    chunks_per_w = rows_per_w // _CHUNK
    rw = table.shape[1]

    mesh = plsc.VectorSubcoreMesh(
        core_axis_name="c", subcore_axis_name="s",
        num_cores=_NC, num_subcores=_NS)

    assert chunks_per_w % _NBUF == 0
    n_steps = chunks_per_w // _NBUF

    @functools.partial(
        pl.kernel,
        out_type=jax.ShapeDtypeStruct((n_rows, _EMBED), jnp.float32),
        mesh=mesh,
        scratch_types=[
            pltpu.VMEM((rows_per_w,), jnp.int32),
            pltpu.VMEM((_NBUF, _CHUNK, rw), jnp.float32),
        ] + [pltpu.SemaphoreType.DMA] * (2 * _NBUF),
        compiler_params=pltpu.CompilerParams(use_tc_tiling_on_sc=False),
    )
    def gather_kernel(idx_hbm, table_hbm, out_hbm, idx_v, rows_v, *sems):
        gsems, wsems = sems[:_NBUF], sems[_NBUF:]
        wid = lax.axis_index("s") * _NC + lax.axis_index("c")
        row0 = wid * rows_per_w

        pltpu.sync_copy(idx_hbm.at[pl.ds(row0, rows_per_w)], idx_v)

        def fire_gather(c, buf):
            pltpu.async_copy(
                table_hbm.at[idx_v.at[pl.ds(c * _CHUNK, _CHUNK)]],
                rows_v.at[buf], gsems[buf])

        def wait_gather(buf):
            pltpu.make_async_copy(
                table_hbm.at[idx_v.at[pl.ds(0, _CHUNK)]],
                rows_v.at[buf], gsems[buf]).wait()

        def wsrc(buf):
            if rw == _EMBED:
                return rows_v.at[buf]
            return rows_v.at[buf].at[:, pl.ds(0, _EMBED)]

        def fire_write(c, buf):
            pltpu.async_copy(
                wsrc(buf),
                out_hbm.at[pl.ds(row0 + c * _CHUNK, _CHUNK)], wsems[buf])

        def wait_write(buf):
            pltpu.make_async_copy(
                wsrc(buf), out_hbm.at[pl.ds(0, _CHUNK)], wsems[buf]).wait()

        for b in range(_NBUF - 1):
            fire_gather(b, b)

        def step_body(step, carry):
            for j in range(_NBUF):
                c = step * _NBUF + j
                fb = (j - 1) % _NBUF

                @pl.when(c >= 1)
                def _():
                    wait_write(fb)

                @pl.when(c + _NBUF - 1 < chunks_per_w)
                def _():
                    fire_gather(c + _NBUF - 1, fb)

                wait_gather(j)
                fire_write(c, j)
            return carry

        lax.fori_loop(0, n_steps, step_body, 0)
        wait_write(_NBUF - 1)

    return gather_kernel(idx_flat, table)


def kernel(lex_indices, func_indices, table):
    b, l = lex_indices.shape
    n_rows = b * l * 2
    lex_t = lex_indices.T.reshape(-1)
    func_t = func_indices.T.reshape(-1)
    idx = jnp.stack([lex_t, func_t], axis=1)
    idx_flat = idx.astype(jnp.int32).reshape(n_rows)
    tab_t = jnp.transpose(table)
    tail_t = jnp.pad(
        lax.slice(tab_t, (0, _NFULL * _BLK), tab_t.shape),
        ((0, 0), (0, _BLK - _TAIL)))
    tab_rm = _sc_relayout(tab_t, tail_t).reshape(table.shape)
    flat = _sc_gather(idx_flat, tab_rm, n_rows=n_rows)
    out_lb = flat.reshape(l, b, 2 * table.shape[1])
    return jnp.transpose(out_lb, (1, 0, 2))


# speedup vs baseline: 1.3680x; 1.3680x over previous
import functools

import jax
import jax.numpy as jnp
from jax import lax
from jax.experimental import pallas as pl
from jax.experimental.pallas import tpu as pltpu
from jax.experimental.pallas import tpu_sc as plsc

_NC = 2
_NS = 16
_NW = _NC * _NS

_EMBED = 64
_CHUNK = 256
_NBUF = 2

_VOCAB = 1000000
_BLK = 128
_NFULL = _VOCAB // _BLK
_TAIL = _VOCAB - _NFULL * _BLK
_COMPUTE_ON = True
_STG_W = 130


def _sc_relayout(tab_t, tail_t):
    jmax = 246
    n_pairs = jmax // 2

    mesh = plsc.VectorSubcoreMesh(
        core_axis_name="c", subcore_axis_name="s",
        num_cores=_NC, num_subcores=_NS)

    @functools.partial(
        pl.kernel,
        out_type=jax.ShapeDtypeStruct((_VOCAB // 2, 128), jnp.float32),
        mesh=mesh,
        scratch_types=[
            pltpu.VMEM((64, _BLK), jnp.float32),
            pltpu.VMEM((64, _BLK), jnp.float32),
            pltpu.VMEM((64, _STG_W), jnp.float32),
            pltpu.VMEM((64, _STG_W), jnp.float32),
        ] + [pltpu.SemaphoreType.DMA] * 4,
        compiler_params=pltpu.CompilerParams(
            use_tc_tiling_on_sc=True, needs_layout_passes=False),
    )
    def relayout_kernel(t_hbm, tail_hbm, out_hbm, in_a, in_b, st_a, st_b,
                        gsem_a, gsem_b, wsem_a, wsem_b):
        wid = lax.axis_index("s") * _NC + lax.axis_index("c")
        ins = (in_a, in_b)
        sts = (st_a, st_b)
        gsems = (gsem_a, gsem_b)
        wsems = (wsem_a, wsem_b)

        iota = lax.iota(jnp.int32, 16)
        half = lax.shift_right_logical(iota, 1)
        cbase = (iota & 1) * 64
        rbases = [8 * g + half for g in range(8)]

        def blk_of(j):
            return wid + _NW * j

        def fire_in(j, buf):
            pltpu.async_copy(
                t_hbm.at[:, pl.ds(blk_of(j) * _BLK, _BLK)],
                ins[buf], gsems[buf])

        def wait_in(buf):
            pltpu.make_async_copy(
                t_hbm.at[:, pl.ds(0, _BLK)], ins[buf], gsems[buf]).wait()

        def fire_out(j, buf):
            pltpu.async_copy(
                sts[buf].at[:, pl.ds(0, 128)],
                out_hbm.at[pl.ds(blk_of(j) * 64, 64)], wsems[buf])

        def wait_out(buf):
            pltpu.make_async_copy(
                sts[buf].at[:, pl.ds(0, 128)],
                out_hbm.at[pl.ds(0, 64)], wsems[buf]).wait()

        def compute(buf, ngroups):
            v_in, st = ins[buf], sts[buf]

            if _COMPUTE_ON:
                @plsc.parallel_loop(0, 64, unroll=4)
                def _(f):
                    c_idx = cbase + f
                    for g in range(ngroups):
                        v = v_in[f, pl.ds(16 * g, 16)]
                        plsc.store_scatter(st, [rbases[g], c_idx], v)

        def do_j(j, buf):
            valid = blk_of(j) < _NFULL
            valid_next = blk_of(j + 1) < _NFULL

            @pl.when(valid_next)
            def _():
                fire_in(j + 1, 1 - buf)

            @pl.when(valid)
            def _():
                wait_in(buf)

                @pl.when(j >= 2)
                def _():
                    wait_out(buf)

                compute(buf, 8)
                fire_out(j, buf)

        fire_in(0, 0)

        def pair(s, carry):
            do_j(2 * s, 0)
            do_j(2 * s + 1, 1)
            return carry

        lax.fori_loop(0, n_pairs, pair, 0)

        wait_out(0)
        wait_out(1)

        @pl.when(wid == _NW - 1)
        def _():
            pltpu.sync_copy(tail_hbm, in_a)

            def frow_t(f, carry):
                c_idx = cbase + f
                for g in range(_TAIL // 16):
                    v = in_a[f, pl.ds(16 * g, 16)]
                    plsc.store_scatter(st_a, [rbases[g], c_idx], v)
                return carry

            lax.fori_loop(0, 64, frow_t, 0)
            pltpu.sync_copy(
                st_a.at[pl.ds(0, _TAIL // 2), pl.ds(0, 128)],
                out_hbm.at[pl.ds(_NFULL * 64, _TAIL // 2)])

    return relayout_kernel(tab_t, tail_t)


def _sc_gather(idx_flat, table, *, n_rows):
    rows_per_w = n_rows // _NW
    chunks_per_w = rows_per_w // _CHUNK
    rw = table.shape[1]

    mesh = plsc.VectorSubcoreMesh(
        core_axis_name="c", subcore_axis_name="s",
        num_cores=_NC, num_subcores=_NS)

    assert chunks_per_w % _NBUF == 0
    n_steps = chunks_per_w // _NBUF

    @functools.partial(
        pl.kernel,
        out_type=jax.ShapeDtypeStruct((n_rows, _EMBED), jnp.float32),
        mesh=mesh,
        scratch_types=[
            pltpu.VMEM((rows_per_w,), jnp.int32),
            pltpu.VMEM((_NBUF, _CHUNK, rw), jnp.float32),
        ] + [pltpu.SemaphoreType.DMA] * (2 * _NBUF),
        compiler_params=pltpu.CompilerParams(use_tc_tiling_on_sc=False),
    )
    def gather_kernel(idx_hbm, table_hbm, out_hbm, idx_v, rows_v, *sems):
        gsems, wsems = sems[:_NBUF], sems[_NBUF:]
        wid = lax.axis_index("s") * _NC + lax.axis_index("c")
        row0 = wid * rows_per_w

        pltpu.sync_copy(idx_hbm.at[pl.ds(row0, rows_per_w)], idx_v)

        def fire_gather(c, buf):
            pltpu.async_copy(
                table_hbm.at[idx_v.at[pl.ds(c * _CHUNK, _CHUNK)]],
                rows_v.at[buf], gsems[buf])

        def wait_gather(buf):
            pltpu.make_async_copy(
                table_hbm.at[idx_v.at[pl.ds(0, _CHUNK)]],
                rows_v.at[buf], gsems[buf]).wait()

        def wsrc(buf):
            if rw == _EMBED:
                return rows_v.at[buf]
            return rows_v.at[buf].at[:, pl.ds(0, _EMBED)]

        def fire_write(c, buf):
            pltpu.async_copy(
                wsrc(buf),
                out_hbm.at[pl.ds(row0 + c * _CHUNK, _CHUNK)], wsems[buf])

        def wait_write(buf):
            pltpu.make_async_copy(
                wsrc(buf), out_hbm.at[pl.ds(0, _CHUNK)], wsems[buf]).wait()

        for b in range(_NBUF - 1):
            fire_gather(b, b)

        def step_body(step, carry):
            for j in range(_NBUF):
                c = step * _NBUF + j
                fb = (j - 1) % _NBUF

                @pl.when(c >= 1)
                def _():
                    wait_write(fb)

                @pl.when(c + _NBUF - 1 < chunks_per_w)
                def _():
                    fire_gather(c + _NBUF - 1, fb)

                wait_gather(j)
                fire_write(c, j)
            return carry

        lax.fori_loop(0, n_steps, step_body, 0)
        wait_write(_NBUF - 1)

    return gather_kernel(idx_flat, table)


def kernel(lex_indices, func_indices, table):
    b, l = lex_indices.shape
    n_rows = b * l * 2
    lex_t = lex_indices.T.reshape(-1)
    func_t = func_indices.T.reshape(-1)
    idx = jnp.stack([lex_t, func_t], axis=1)
    idx_flat = idx.astype(jnp.int32).reshape(n_rows)
    tab_t = jnp.transpose(table)
    tail_t = jnp.pad(
        lax.slice(tab_t, (0, _NFULL * _BLK), tab_t.shape),
        ((0, 0), (0, _BLK - _TAIL)))
    tab_rm = _sc_relayout(tab_t, tail_t).reshape(table.shape)
    flat = _sc_gather(idx_flat, tab_rm, n_rows=n_rows)
    out_lb = flat.reshape(l, b, 2 * table.shape[1])
    return jnp.transpose(out_lb, (1, 0, 2))


# speedup vs baseline: 1.8544x; 1.3556x over previous
import functools

import jax
import jax.numpy as jnp
from jax import lax
from jax.experimental import pallas as pl
from jax.experimental.pallas import tpu as pltpu
from jax.experimental.pallas import tpu_sc as plsc

_NC = 2
_NS = 16
_NW = _NC * _NS

_EMBED = 64
_CHUNK = 256
_NBUF = 2

def _sc_gather(idx_flat, table, *, n_rows):
    rows_per_w = n_rows // _NW
    chunks_per_w = rows_per_w // _CHUNK
    rw = table.shape[1]

    mesh = plsc.VectorSubcoreMesh(
        core_axis_name="c", subcore_axis_name="s",
        num_cores=_NC, num_subcores=_NS)

    assert chunks_per_w % _NBUF == 0
    n_steps = chunks_per_w // _NBUF

    @functools.partial(
        pl.kernel,
        out_type=jax.ShapeDtypeStruct((n_rows, _EMBED), jnp.float32),
        mesh=mesh,
        scratch_types=[
            pltpu.VMEM((rows_per_w,), jnp.int32),
            pltpu.VMEM((_NBUF, _CHUNK, rw), jnp.float32),
        ] + [pltpu.SemaphoreType.DMA] * (2 * _NBUF),
        compiler_params=pltpu.CompilerParams(use_tc_tiling_on_sc=False),
    )
    def gather_kernel(idx_hbm, table_hbm, out_hbm, idx_v, rows_v, *sems):
        gsems, wsems = sems[:_NBUF], sems[_NBUF:]
        wid = lax.axis_index("s") * _NC + lax.axis_index("c")
        row0 = wid * rows_per_w

        pltpu.sync_copy(idx_hbm.at[pl.ds(row0, rows_per_w)], idx_v)

        def fire_gather(c, buf):
            pltpu.async_copy(
                table_hbm.at[idx_v.at[pl.ds(c * _CHUNK, _CHUNK)]],
                rows_v.at[buf], gsems[buf])

        def wait_gather(buf):
            pltpu.make_async_copy(
                table_hbm.at[idx_v.at[pl.ds(0, _CHUNK)]],
                rows_v.at[buf], gsems[buf]).wait()

        def wsrc(buf):
            if rw == _EMBED:
                return rows_v.at[buf]
            return rows_v.at[buf].at[:, pl.ds(0, _EMBED)]

        def fire_write(c, buf):
            pltpu.async_copy(
                wsrc(buf),
                out_hbm.at[pl.ds(row0 + c * _CHUNK, _CHUNK)], wsems[buf])

        def wait_write(buf):
            pltpu.make_async_copy(
                wsrc(buf), out_hbm.at[pl.ds(0, _CHUNK)], wsems[buf]).wait()

        for b in range(_NBUF - 1):
            fire_gather(b, b)

        def step_body(step, carry):
            for j in range(_NBUF):
                c = step * _NBUF + j
                fb = (j - 1) % _NBUF

                @pl.when(c >= 1)
                def _():
                    wait_write(fb)

                @pl.when(c + _NBUF - 1 < chunks_per_w)
                def _():
                    fire_gather(c + _NBUF - 1, fb)

                wait_gather(j)
                fire_write(c, j)
            return carry

        lax.fori_loop(0, n_steps, step_body, 0)
        wait_write(_NBUF - 1)

    return gather_kernel(idx_flat, table)


def kernel(lex_indices, func_indices, table):
    b, l = lex_indices.shape
    n_rows = b * l * 2
    lex_t = lex_indices.T.reshape(-1)
    func_t = func_indices.T.reshape(-1)
    idx = jnp.stack([lex_t, func_t], axis=1)
    idx_flat = idx.astype(jnp.int32).reshape(n_rows)
    tab_p = jnp.pad(table, ((0, 0), (0, 2 * _EMBED - table.shape[1])))
    flat = _sc_gather(idx_flat, tab_p, n_rows=n_rows)
    out_lb = flat.reshape(l, b, 2 * table.shape[1])
    return jnp.transpose(out_lb, (1, 0, 2))
